# baseline (speedup 1.0000x reference)
import jax
import jax.numpy as jnp
from jax import lax
from jax.experimental import pallas as pl
from jax.experimental.pallas import tpu as pltpu
from jax.experimental.pallas import tpu_sc as plsc

VOCAB = 100000
D = 1024
B = 4
SEQ = 2048
N_TOK = B * SEQ

NC = 2
NS = 16
NW = NC * NS
LANES = 16

POS_PER_W = SEQ // NW
C = 16
NPC = POS_PER_W // C
NSTEP = NPC * B
NOBUF = 4
NGBUF = 2


def _body(ids_hbm, table_hbm, pe_hbm, out_hbm,
          idx_all, o0, o1, o2, o3, g0, g1,
          f0, f1, f2, f3, gs0, gs1, ss0, ss1, ss2, ss3):
    c = lax.axis_index("c")
    s = lax.axis_index("s")
    wid = s * NC + c
    wpos = wid * POS_PER_W

    obuf = [o0, o1, o2, o3]
    gbuf = [g0, g1]
    fsem = [f0, f1, f2, f3]
    gsem = [gs0, gs1]
    ssem = [ss0, ss1, ss2, ss3]

    for b in range(B):
        pltpu.sync_copy(ids_hbm.at[pl.ds(b * SEQ + wpos, POS_PER_W)],
                        idx_all.at[b])

    def fire_pefill(step):
        pc = step // B
        q = step % NOBUF
        return pltpu.async_copy(
            pe_hbm.at[pl.ds(wpos + pc * C, C)], obuf[q], fsem[q])

    def fire_gather(step):
        pc, b = step // B, step % B
        p = step % NGBUF
        return pltpu.async_copy(
            table_hbm.at[idx_all.at[b, pl.ds(pc * C, C)]], gbuf[p], gsem[p])

    fills = {0: fire_pefill(0), 1: fire_pefill(1)}
    gathers = {0: fire_gather(0)}
    stores = {}

    for step in range(NSTEP):
        pc, b = step // B, step % B
        q = step % NOBUF
        p = step % NGBUF
        if step + 2 < NSTEP:
            if step - 2 >= 0:
                stores[step - 2].wait()
            fills[step + 2] = fire_pefill(step + 2)
        if step + 1 < NSTEP:
            gathers[step + 1] = fire_gather(step + 1)
        gathers[step].wait()
        fills[step].wait()

        gb, ob = gbuf[p], obuf[q]

        @plsc.parallel_loop(0, C, 1)
        def row_body(r, gb=gb, ob=ob):
            for k in range(D // LANES):
                sl = pl.ds(k * LANES, LANES)
                plsc.addupdate(ob.at[r, sl], gb[r, sl])

        t = b * SEQ + wpos + pc * C
        stores[step] = pltpu.async_copy(ob, out_hbm.at[pl.ds(t, C)], ssem[q])

    stores[NSTEP - 2].wait()
    stores[NSTEP - 1].wait()


def kernel(input_ids, table, pe):
    ids_flat = input_ids.reshape(N_TOK).astype(jnp.int32)
    mesh = plsc.VectorSubcoreMesh(core_axis_name="c", subcore_axis_name="s")
    out = pl.kernel(
        _body,
        mesh=mesh,
        out_type=jax.ShapeDtypeStruct((N_TOK, D), jnp.float32),
        scratch_types=[
            pltpu.VMEM((B, POS_PER_W), jnp.int32),
            pltpu.VMEM((C, D), jnp.float32),
            pltpu.VMEM((C, D), jnp.float32),
            pltpu.VMEM((C, D), jnp.float32),
            pltpu.VMEM((C, D), jnp.float32),
            pltpu.VMEM((C, D), jnp.float32),
            pltpu.VMEM((C, D), jnp.float32),
            pltpu.SemaphoreType.DMA,
            pltpu.SemaphoreType.DMA,
            pltpu.SemaphoreType.DMA,
            pltpu.SemaphoreType.DMA,
            pltpu.SemaphoreType.DMA,
            pltpu.SemaphoreType.DMA,
            pltpu.SemaphoreType.DMA,
            pltpu.SemaphoreType.DMA,
            pltpu.SemaphoreType.DMA,
            pltpu.SemaphoreType.DMA,
        ],
    )(ids_flat, table, pe)
    return out.reshape(B, SEQ, D)


# speedup vs baseline: 1.0335x; 1.0335x over previous
import jax
import jax.numpy as jnp
from jax import lax
from jax.experimental import pallas as pl
from jax.experimental.pallas import tpu as pltpu
from jax.experimental.pallas import tpu_sc as plsc

VOCAB = 100000
D = 1024
B = 4
SEQ = 2048
N_TOK = B * SEQ

NC = 2
NS = 16
NW = NC * NS
LANES = 16

POS_PER_W = SEQ // NW
CHUNKS = (16, 16, 16, 16)
OFFS = (0, 16, 32, 48)
CMAX = 16
NPC = len(CHUNKS)
NSTEP = NPC * B


def _body(ids_hbm, table_hbm, pe_hbm, out_hbm,
          pe_v, idx_all, g0, g1, o0, o1, sg0, sg1, ss0, ss1, psem):
    c = lax.axis_index("c")
    s = lax.axis_index("s")
    wid = s * NC + c
    wpos = wid * POS_PER_W

    gbuf = [g0, g1]
    obuf = [o0, o1]
    gsem = [sg0, sg1]
    ssem = [ss0, ss1]

    def pe_fetch(pc):
        return pltpu.async_copy(
            pe_hbm.at[pl.ds(wpos + OFFS[pc], CHUNKS[pc])],
            pe_v.at[pl.ds(0, CHUNKS[pc])], psem)

    pe_cps = {0: pe_fetch(0)}
    for st in range(NSTEP):
        pcs, bs = st // B, st % B
        pltpu.sync_copy(
            ids_hbm.at[pl.ds(bs * SEQ + wpos + OFFS[pcs], CHUNKS[pcs])],
            idx_all.at[st, pl.ds(0, CHUNKS[pcs])])

    def fire_gather(step):
        pc, b = step // B, step % B
        cs = CHUNKS[pc]
        p = step % 2
        return pltpu.async_copy(
            table_hbm.at[idx_all.at[step]],
            gbuf[p], gsem[p])

    gathers = {0: fire_gather(0)}
    stores = {}

    for step in range(NSTEP):
        pc, b = step // B, step % B
        cs = CHUNKS[pc]
        p = step % 2
        gathers[step].wait()
        if step + 1 < NSTEP:
            gathers[step + 1] = fire_gather(step + 1)
        if b == 0:
            pe_cps[pc].wait()
        if step - 2 >= 0:
            stores[step - 2].wait()

        gb, ob = gbuf[p], obuf[p]

        @plsc.parallel_loop(0, cs, 1)
        def row_body(r, gb=gb, ob=ob):
            for k in range(D // LANES):
                sl = pl.ds(k * LANES, LANES)
                ob[r, sl] = gb[r, sl] + pe_v[r, sl]

        t = b * SEQ + wpos + OFFS[pc]
        stores[step] = pltpu.async_copy(
            ob.at[pl.ds(0, cs)], out_hbm.at[pl.ds(t, cs)], ssem[p])

        if b == B - 1 and pc + 1 < NPC:
            pe_cps[pc + 1] = pe_fetch(pc + 1)

    stores[NSTEP - 2].wait()
    stores[NSTEP - 1].wait()


def kernel(input_ids, table, pe):
    ids_flat = input_ids.reshape(N_TOK).astype(jnp.int32)
    mesh = plsc.VectorSubcoreMesh(core_axis_name="c", subcore_axis_name="s")
    out = pl.kernel(
        _body,
        mesh=mesh,
        out_type=jax.ShapeDtypeStruct((N_TOK, D), jnp.float32),
        scratch_types=[
            pltpu.VMEM((CMAX, D), jnp.float32),
            pltpu.VMEM((NSTEP, CMAX), jnp.int32),
            pltpu.VMEM((CMAX, D), jnp.float32),
            pltpu.VMEM((CMAX, D), jnp.float32),
            pltpu.VMEM((CMAX, D), jnp.float32),
            pltpu.VMEM((CMAX, D), jnp.float32),
            pltpu.SemaphoreType.DMA,
            pltpu.SemaphoreType.DMA,
            pltpu.SemaphoreType.DMA,
            pltpu.SemaphoreType.DMA,
            pltpu.SemaphoreType.DMA,
        ],
    )(ids_flat, table, pe)
    return out.reshape(B, SEQ, D)


# speedup vs baseline: 1.1505x; 1.1133x over previous
import jax
import jax.numpy as jnp
from jax import lax
from jax.experimental import pallas as pl
from jax.experimental.pallas import tpu as pltpu
from jax.experimental.pallas import tpu_sc as plsc

VOCAB = 100000
D = 1024
B = 4
SEQ = 2048
N_TOK = B * SEQ

NC = 2
NS = 16
NW = NC * NS
LANES = 16

POS_PER_W = SEQ // NW
CHUNKS = (24, 24, 16)
OFFS = (0, 24, 48)
CMAX = 24
NPC = len(CHUNKS)
NSTEP = NPC * B


def _body(ids_hbm, table_hbm, pe_hbm, out_hbm,
          pe_v, idx_all, g0, g1, o0, o1, sg0, sg1, ss0, ss1, psem):
    c = lax.axis_index("c")
    s = lax.axis_index("s")
    wid = s * NC + c
    wpos = wid * POS_PER_W

    gbuf = [g0, g1]
    obuf = [o0, o1]
    gsem = [sg0, sg1]
    ssem = [ss0, ss1]

    def pe_fetch(pc):
        return pltpu.async_copy(
            pe_hbm.at[pl.ds(wpos + OFFS[pc], CHUNKS[pc])],
            pe_v.at[pl.ds(0, CHUNKS[pc])], psem)

    pe_cps = {0: pe_fetch(0)}
    for b in range(B):
        pltpu.sync_copy(ids_hbm.at[pl.ds(b * SEQ + wpos, POS_PER_W)],
                        idx_all.at[b])

    def fire_gather(step):
        pc, b = step // B, step % B
        cs = CHUNKS[pc]
        p = step % 2
        return pltpu.async_copy(
            table_hbm.at[idx_all.at[b, pl.ds(OFFS[pc], cs)]],
            gbuf[p].at[pl.ds(0, cs)], gsem[p])

    gathers = {0: fire_gather(0)}
    stores = {}

    for step in range(NSTEP):
        pc, b = step // B, step % B
        cs = CHUNKS[pc]
        p = step % 2
        gathers[step].wait()
        if step + 1 < NSTEP:
            gathers[step + 1] = fire_gather(step + 1)
        if b == 0:
            pe_cps[pc].wait()
        if step - 2 >= 0:
            stores[step - 2].wait()

        gb, ob = gbuf[p], obuf[p]

        @plsc.parallel_loop(0, cs, 1)
        def row_body(r, gb=gb, ob=ob):
            for k in range(D // LANES):
                sl = pl.ds(k * LANES, LANES)
                ob[r, sl] = gb[r, sl] + pe_v[r, sl]

        t = b * SEQ + wpos + OFFS[pc]
        stores[step] = pltpu.async_copy(
            ob.at[pl.ds(0, cs)], out_hbm.at[pl.ds(t, cs)], ssem[p])

        if b == B - 1 and pc + 1 < NPC:
            pe_cps[pc + 1] = pe_fetch(pc + 1)

    stores[NSTEP - 2].wait()
    stores[NSTEP - 1].wait()


def kernel(input_ids, table, pe):
    ids_flat = input_ids.reshape(N_TOK).astype(jnp.int32)
    mesh = plsc.VectorSubcoreMesh(core_axis_name="c", subcore_axis_name="s")
    out = pl.kernel(
        _body,
        mesh=mesh,
        out_type=jax.ShapeDtypeStruct((N_TOK, D), jnp.float32),
        scratch_types=[
            pltpu.VMEM((CMAX, D), jnp.float32),
            pltpu.VMEM((B, POS_PER_W), jnp.int32),
            pltpu.VMEM((CMAX, D), jnp.float32),
            pltpu.VMEM((CMAX, D), jnp.float32),
            pltpu.VMEM((CMAX, D), jnp.float32),
            pltpu.VMEM((CMAX, D), jnp.float32),
            pltpu.SemaphoreType.DMA,
            pltpu.SemaphoreType.DMA,
            pltpu.SemaphoreType.DMA,
            pltpu.SemaphoreType.DMA,
            pltpu.SemaphoreType.DMA,
        ],
    )(ids_flat, table, pe)
    return out.reshape(B, SEQ, D)
